# Initial kernel scaffold; baseline (speedup 1.0000x reference)
import jax
import jax.numpy as jnp
from jax.experimental import pallas as pl


def kernel(x, emb_w, lin_w, bias):
    raise NotImplementedError("write your pallas kernel here")


# speedup vs baseline: 2.0258x; 2.0258x over previous
import dataclasses
import functools

import jax
import jax.numpy as jnp
from jax.experimental import pallas as pl
from jax.experimental.pallas import tpu as pltpu
from jax.experimental.pallas import tpu_sc as plsc

B = 16384
F = 26
D = 32
L = 16
C = 16
W = 104
IPS = C * F
GPS = IPS // W
NSTEPS = B // C


def _fm_step(emb_hbm, lin_hbm, emb_buf, lin_buf, bias_buf, u_buf, sem,
             idx_vmem, out_vmem):
    cps = []
    for g in range(GPS):
        cps.append(pltpu.async_copy(
            emb_hbm.at[idx_vmem.at[g]], emb_buf.at[pl.ds(g * W, W)], sem))
        cps.append(pltpu.async_copy(
            lin_hbm.at[idx_vmem.at[g]], lin_buf.at[pl.ds(g * W, W)], sem))
    for cp in cps:
        cp.wait()
    for c in range(C):
        s0 = emb_buf[c, pl.ds(0, L)]
        s1 = emb_buf[c, pl.ds(L, L)]
        q0 = s0 * s0
        q1 = s1 * s1
        for f in range(1, F):
            v0 = emb_buf[f * C + c, pl.ds(0, L)]
            v1 = emb_buf[f * C + c, pl.ds(L, L)]
            s0 = s0 + v0
            s1 = s1 + v1
            q0 = q0 + v0 * v0
            q1 = q1 + v1 * v1
        u_buf[pl.ds(c * D, L)] = s0 * s0 - q0
        u_buf[pl.ds(c * D + L, L)] = s1 * s1 - q1
    lanes = jax.lax.iota(jnp.int32, L)
    rowsel = lanes * D
    acc = plsc.load_gather(u_buf, [rowsel])
    for d in range(1, D):
        acc = acc + plsc.load_gather(u_buf, [rowsel + d])
    lin = lin_buf[pl.ds(0, L)]
    for f in range(1, F):
        lin = lin + lin_buf[pl.ds(f * C, L)]
    out = 0.5 * acc + lin + bias_buf[...]
    out = jnp.minimum(jnp.maximum(out, -2.0), 2.0)
    out_vmem[0, pl.ds(0, L)] = out


def kernel(x, emb_w, lin_w, bias):
    idx = (x.astype(jnp.int32)
           .reshape(NSTEPS, C, F)
           .transpose(0, 2, 1)
           .reshape(B * F // W, W))
    lin_flat = lin_w.reshape(-1)
    bias16 = jnp.broadcast_to(bias, (L,))
    mesh = plsc.VectorSubcoreMesh(core_axis_name="core",
                                  subcore_axis_name="subcore")
    cp = pltpu.CompilerParams(use_tc_tiling_on_sc=False)
    if "needs_layout_passes" in pltpu.CompilerParams.__dataclass_fields__:
        cp = dataclasses.replace(cp, needs_layout_passes=False)

    @functools.partial(
        pl.kernel,
        out_type=jax.ShapeDtypeStruct((NSTEPS, C), jnp.float32),
        mesh=mesh,
        compiler_params=cp,
        scratch_types=[
            pltpu.VMEM((IPS, D), jnp.float32),
            pltpu.VMEM((IPS,), jnp.float32),
            pltpu.VMEM((L,), jnp.float32),
            pltpu.VMEM((C * D,), jnp.float32),
            pltpu.SemaphoreType.DMA,
        ],
    )
    def run(idx_hbm, emb_hbm, lin_hbm, bias_hbm, out_hbm,
            emb_buf, lin_buf, bias_buf, u_buf, sem):
        pltpu.sync_copy(bias_hbm, bias_buf)
        body = functools.partial(_fm_step, emb_hbm, lin_hbm,
                                 emb_buf, lin_buf, bias_buf, u_buf, sem)
        pltpu.emit_pipeline(
            body,
            grid=(NSTEPS,),
            in_specs=[pl.BlockSpec((GPS, W), lambda i: (i, 0))],
            out_specs=[pl.BlockSpec((1, C), lambda i: (i, 0))],
            core_axis_name=("core", "subcore"),
            dimension_semantics=(pltpu.PARALLEL,),
        )(idx_hbm, out_hbm)

    out = run(idx, emb_w, lin_flat, bias16)
    return out.reshape(B)


# speedup vs baseline: 2.1183x; 1.0457x over previous
import dataclasses
import functools

import jax
import jax.numpy as jnp
from jax.experimental import pallas as pl
from jax.experimental.pallas import tpu as pltpu
from jax.experimental.pallas import tpu_sc as plsc

B = 16384
F = 26
D = 32
L = 16
C = 64
W = 128
IPS = C * F
GPS = IPS // W
NSTEPS = B // C


def _fm_step(emb_hbm, lin_hbm, emb_buf, lin_buf, bias_buf, u_buf, sem,
             idx_vmem, out_vmem):
    cps = []
    for g in range(GPS):
        cps.append(pltpu.async_copy(
            emb_hbm.at[idx_vmem.at[g]], emb_buf.at[pl.ds(g * W, W)], sem))
        cps.append(pltpu.async_copy(
            lin_hbm.at[idx_vmem.at[g]], lin_buf.at[pl.ds(g * W, W)], sem))
    for cp in cps:
        cp.wait()

    @pl.loop(0, C)
    def _(c):
        s0 = emb_buf[c, pl.ds(0, L)]
        s1 = emb_buf[c, pl.ds(L, L)]
        q0 = s0 * s0
        q1 = s1 * s1
        for f in range(1, F):
            v0 = emb_buf[f * C + c, pl.ds(0, L)]
            v1 = emb_buf[f * C + c, pl.ds(L, L)]
            s0 = s0 + v0
            s1 = s1 + v1
            q0 = q0 + v0 * v0
            q1 = q1 + v1 * v1
        u_buf[pl.ds(c * D, L)] = s0 * s0 - q0
        u_buf[pl.ds(c * D + L, L)] = s1 * s1 - q1

    lanes = jax.lax.iota(jnp.int32, L)
    for t in range(C // L):
        rowsel = (lanes + t * L) * D
        acc = plsc.load_gather(u_buf, [rowsel])
        for d in range(1, D):
            acc = acc + plsc.load_gather(u_buf, [rowsel + d])
        lin = lin_buf[pl.ds(t * L, L)]
        for f in range(1, F):
            lin = lin + lin_buf[pl.ds(f * C + t * L, L)]
        out = 0.5 * acc + lin + bias_buf[...]
        out = jnp.minimum(jnp.maximum(out, -2.0), 2.0)
        out_vmem[0, pl.ds(t * L, L)] = out


def kernel(x, emb_w, lin_w, bias):
    idx = (x.astype(jnp.int32)
           .reshape(NSTEPS, C, F)
           .transpose(0, 2, 1)
           .reshape(B * F // W, W))
    lin_flat = lin_w.reshape(-1)
    bias16 = jnp.broadcast_to(bias, (L,))
    mesh = plsc.VectorSubcoreMesh(core_axis_name="core",
                                  subcore_axis_name="subcore")
    cp = pltpu.CompilerParams(use_tc_tiling_on_sc=False)
    if "needs_layout_passes" in pltpu.CompilerParams.__dataclass_fields__:
        cp = dataclasses.replace(cp, needs_layout_passes=False)

    @functools.partial(
        pl.kernel,
        out_type=jax.ShapeDtypeStruct((NSTEPS, C), jnp.float32),
        mesh=mesh,
        compiler_params=cp,
        scratch_types=[
            pltpu.VMEM((IPS, D), jnp.float32),
            pltpu.VMEM((IPS,), jnp.float32),
            pltpu.VMEM((L,), jnp.float32),
            pltpu.VMEM((C * D,), jnp.float32),
            pltpu.SemaphoreType.DMA,
        ],
    )
    def run(idx_hbm, emb_hbm, lin_hbm, bias_hbm, out_hbm,
            emb_buf, lin_buf, bias_buf, u_buf, sem):
        pltpu.sync_copy(bias_hbm, bias_buf)
        body = functools.partial(_fm_step, emb_hbm, lin_hbm,
                                 emb_buf, lin_buf, bias_buf, u_buf, sem)
        pltpu.emit_pipeline(
            body,
            grid=(NSTEPS,),
            in_specs=[pl.BlockSpec((GPS, W), lambda i: (i, 0))],
            out_specs=[pl.BlockSpec((1, C), lambda i: (i, 0))],
            core_axis_name=("core", "subcore"),
            dimension_semantics=(pltpu.PARALLEL,),
        )(idx_hbm, out_hbm)

    out = run(idx, emb_w, lin_flat, bias16)
    return out.reshape(B)
